# baseline (speedup 1.0000x reference)
import functools

import jax
import jax.numpy as jnp
from jax import lax
from jax.experimental import pallas as pl
from jax.experimental.pallas import tpu as pltpu
from jax.experimental.pallas import tpu_sc as plsc

D = 128
LANG_PAD = 104


def _project_body(char_ref, lang_ref, w_ref, b_ref, cout_ref, lout_ref):
    w = w_ref[...]
    w1 = w[:, :D]
    w2 = w[:, D:]
    cn = (((1,), (1,)), ((), ()))
    cout_ref[...] = (
        lax.dot_general(char_ref[...], w1, cn, preferred_element_type=jnp.float32)
        + b_ref[...]
    )
    lout_ref[...] = lax.dot_general(
        lang_ref[...], w2, cn, preferred_element_type=jnp.float32
    )


def _project(char_table, lang_table_padded, W, b2d):
    n_chars = char_table.shape[0]
    return pl.pallas_call(
        _project_body,
        out_shape=[
            jax.ShapeDtypeStruct((n_chars, D), jnp.float32),
            jax.ShapeDtypeStruct((LANG_PAD, D), jnp.float32),
        ],
    )(char_table, lang_table_padded, W, b2d)


def _make_sc_gather(n_total):
    info = plsc.get_sparse_core_info()
    nw = info.num_cores * info.num_subcores
    per_w = n_total // nw
    ch = 128
    n_ch = per_w // ch
    mesh = plsc.VectorSubcoreMesh(core_axis_name="c", subcore_axis_name="s")

    @functools.partial(
        pl.kernel,
        mesh=mesh,
        out_type=jax.ShapeDtypeStruct((n_total, D), jnp.float32),
        scratch_types=[
            pltpu.VMEM((LANG_PAD, D), jnp.float32),
            pltpu.VMEM((per_w,), jnp.int32),
            pltpu.VMEM((per_w + 16,), jnp.int32),
            pltpu.VMEM((ch, D), jnp.float32),
            pltpu.VMEM((ch, D), jnp.float32),
            pltpu.VMEM((ch, D), jnp.float32),
            pltpu.VMEM((ch, D), jnp.float32),
            pltpu.SemaphoreType.DMA,
            pltpu.SemaphoreType.DMA,
            pltpu.SemaphoreType.DMA,
            pltpu.SemaphoreType.DMA,
        ],
    )
    def sc_gather(cproj_hbm, lproj_hbm, ci_hbm, li_hbm, out_hbm,
                  lang_v, ci_v, li_v, ga0, ga1, ob0, ob1, gs0, gs1, ss0, ss1):
        wid = lax.axis_index("s") * info.num_cores + lax.axis_index("c")
        base = wid * per_w
        pltpu.sync_copy(lproj_hbm, lang_v)
        pltpu.sync_copy(ci_hbm.at[pl.ds(base, per_w)], ci_v)
        pltpu.sync_copy(li_hbm.at[pl.ds(base, per_w)], li_v.at[pl.ds(0, per_w)])
        gbufs = (ga0, ga1)
        obufs = (ob0, ob1)
        gsems = (gs0, gs1)
        ssems = (ss0, ss1)

        def start_gather(g):
            return pltpu.async_copy(
                cproj_hbm.at[ci_v.at[pl.ds(g * ch, ch)]], gbufs[g % 2], gsems[g % 2]
            )

        pend_gather = {0: start_gather(0), 1: start_gather(1)}
        pend_scatter = {}
        for g in range(n_ch):
            cur = g % 2
            pend_gather.pop(g).wait()
            if g - 2 >= 0:
                pend_scatter.pop(g - 2).wait()
            gb = gbufs[cur]
            ob = obufs[cur]
            lbase = g * ch

            def row(r, carry, gb=gb, ob=ob, lbase=lbase):
                lr = li_v[pl.ds(lbase + r, 16)][0]
                for c in range(0, D, 16):
                    ob[r, pl.ds(c, 16)] = (
                        gb[r, pl.ds(c, 16)] + lang_v[lr, pl.ds(c, 16)]
                    )
                return carry

            lax.fori_loop(0, ch, row, 0)
            pend_scatter[g] = pltpu.async_copy(
                ob, out_hbm.at[pl.ds(base + g * ch, ch)], ssems[cur]
            )
            if g + 2 < n_ch:
                pend_gather[g + 2] = start_gather(g + 2)
        for s in pend_scatter.values():
            s.wait()

    return sc_gather


def kernel(char_indices, lang_indices, char_table, lang_table, W, b):
    B, L = char_indices.shape
    n_total = B * L
    lang_padded = jnp.pad(lang_table, ((0, LANG_PAD - lang_table.shape[0]), (0, 0)))
    cproj, lproj = _project(char_table, lang_padded, W, b.reshape(1, D))
    ci = char_indices.reshape(-1).astype(jnp.int32)
    li = lang_indices.reshape(-1).astype(jnp.int32)
    out = _make_sc_gather(n_total)(cproj, lproj, ci, li)
    return out.reshape(B, L, D)


# speedup vs baseline: 1.1368x; 1.1368x over previous
import functools

import jax
import jax.numpy as jnp
from jax import lax
from jax.experimental import pallas as pl
from jax.experimental.pallas import tpu as pltpu
from jax.experimental.pallas import tpu_sc as plsc

D = 128
LANG_PAD = 104


def _project_body(char_ref, lang_ref, w_ref, b_ref, cout_ref, lout_ref):
    w = w_ref[...]
    w1 = w[:, :D]
    w2 = w[:, D:]
    cn = (((1,), (1,)), ((), ()))
    cout_ref[...] = (
        lax.dot_general(char_ref[...], w1, cn, preferred_element_type=jnp.float32)
        + b_ref[...]
    )
    lout_ref[...] = lax.dot_general(
        lang_ref[...], w2, cn, preferred_element_type=jnp.float32
    )


def _project(char_table, lang_table_padded, W, b2d):
    n_chars = char_table.shape[0]
    return pl.pallas_call(
        _project_body,
        out_shape=[
            jax.ShapeDtypeStruct((n_chars, D), jnp.float32),
            jax.ShapeDtypeStruct((LANG_PAD, D), jnp.float32),
        ],
    )(char_table, lang_table_padded, W, b2d)


def _make_sc_gather(n_total):
    info = plsc.get_sparse_core_info()
    nw = info.num_cores * info.num_subcores
    per_w = n_total // nw
    ch = 128
    n_ch = per_w // ch
    mesh = plsc.VectorSubcoreMesh(core_axis_name="c", subcore_axis_name="s")

    @functools.partial(
        pl.kernel,
        mesh=mesh,
        out_type=jax.ShapeDtypeStruct((n_total, D), jnp.float32),
        scratch_types=[
            pltpu.VMEM((LANG_PAD, D), jnp.float32),
            pltpu.VMEM((per_w,), jnp.int32),
            pltpu.VMEM((per_w + 16,), jnp.int32),
            pltpu.VMEM((ch, D), jnp.float32),
            pltpu.VMEM((ch, D), jnp.float32),
            pltpu.VMEM((ch, D), jnp.float32),
            pltpu.VMEM((ch, D), jnp.float32),
            pltpu.SemaphoreType.DMA,
            pltpu.SemaphoreType.DMA,
            pltpu.SemaphoreType.DMA,
            pltpu.SemaphoreType.DMA,
        ],
    )
    def sc_gather(cproj_hbm, lproj_hbm, ci_hbm, li_hbm, out_hbm,
                  lang_v, ci_v, li_v, ga0, ga1, ob0, ob1, gs0, gs1, ss0, ss1):
        wid = lax.axis_index("s") * info.num_cores + lax.axis_index("c")
        base = wid * per_w
        pltpu.sync_copy(lproj_hbm, lang_v)
        pltpu.sync_copy(ci_hbm.at[pl.ds(base, per_w)], ci_v)
        pltpu.sync_copy(li_hbm.at[pl.ds(base, per_w)], li_v.at[pl.ds(0, per_w)])
        gbufs = (ga0, ga1)
        obufs = (ob0, ob1)
        gsems = (gs0, gs1)
        ssems = (ss0, ss1)

        def start_gather(g, b):
            return pltpu.async_copy(
                cproj_hbm.at[ci_v.at[pl.ds(g * ch, ch)]], gbufs[b], gsems[b]
            )

        def wait_gather(b):
            pltpu.make_async_copy(
                cproj_hbm.at[ci_v.at[pl.ds(0, ch)]], gbufs[b], gsems[b]
            ).wait()

        def start_scatter(g, b):
            return pltpu.async_copy(
                obufs[b], out_hbm.at[pl.ds(base + g * ch, ch)], ssems[b]
            )

        def wait_scatter(b):
            pltpu.make_async_copy(
                obufs[b], out_hbm.at[pl.ds(base, ch)], ssems[b]
            ).wait()

        def add_chunk(g, b):
            gb = gbufs[b]
            ob = obufs[b]

            def grp(t, carry):
                r0 = t * 16
                lvec = li_v[pl.ds(g * ch + r0, 16)]
                for j in range(16):
                    lr = lvec[j]
                    for c in range(0, D, 16):
                        ob[r0 + j, pl.ds(c, 16)] = (
                            gb[r0 + j, pl.ds(c, 16)] + lang_v[lr, pl.ds(c, 16)]
                        )
                return carry

            lax.fori_loop(0, ch // 16, grp, 0)

        start_gather(0, 0)
        start_gather(1, 1)
        for g in (0, 1):
            b = g % 2
            wait_gather(b)
            add_chunk(g, b)
            start_scatter(g, b)
            start_gather(g + 2, b)

        def pair(i, carry):
            g = 2 + 2 * i
            for b in (0, 1):
                wait_gather(b)
                wait_scatter(b)
                add_chunk(g + b, b)
                start_scatter(g + b, b)
                start_gather(g + b + 2, b)
            return carry

        n_pairs = (n_ch - 4) // 2
        lax.fori_loop(0, n_pairs, pair, 0)

        for g in (n_ch - 2, n_ch - 1):
            b = g % 2
            wait_gather(b)
            wait_scatter(b)
            add_chunk(g, b)
            start_scatter(g, b)
        for b in (0, 1):
            wait_scatter(b)

    return sc_gather


def kernel(char_indices, lang_indices, char_table, lang_table, W, b):
    B, L = char_indices.shape
    n_total = B * L
    lang_padded = jnp.pad(lang_table, ((0, LANG_PAD - lang_table.shape[0]), (0, 0)))
    cproj, lproj = _project(char_table, lang_padded, W, b.reshape(1, D))
    ci = char_indices.reshape(-1).astype(jnp.int32)
    li = lang_indices.reshape(-1).astype(jnp.int32)
    out = _make_sc_gather(n_total)(cproj, lproj, ci, li)
    return out.reshape(B, L, D)
